# baseline (device time: 39589 ns/iter reference)
import jax
import jax.numpy as jnp
from jax import lax
from jax.experimental import pallas as pl
from jax.experimental.pallas import tpu as pltpu

N_DEV = 32
H = 512


def _gelu(v):
    c = 0.7978845608028654
    return 0.5 * v * (1.0 + jnp.tanh(c * (v + 0.044715 * v * v * v)))


def kernel(x, w_mat):
    m, k_per = x.shape
    _, n = w_mat.shape
    m_per = m // N_DEV

    def body(x_ref, w_ref, out_ref, partA, partB, rA, rB, rA2, rB2, rX,
             szA, rzA, syB, ryB, syA, ryA, szB, rzB, sX, rXs):
        p = lax.axis_index("i")
        z = p // 8
        j = p % 8
        y = j // 2
        xc = (j + y) % 2
        t = p % 2
        px = p + 1 - 2 * t

        right_z = (p + 8) % N_DEV
        left_z = (p - 8) % N_DEV
        yn = (y + 1) % 4
        yp = (y - 1) % 4
        right_y = 8 * z + 2 * yn + (xc + yn) % 2
        left_y = 8 * z + 2 * yp + (xc + yp) % 2

        def rdma(src, dst, ssem, rsem, tgt):
            r = pltpu.make_async_remote_copy(
                src_ref=src, dst_ref=dst, send_sem=ssem, recv_sem=rsem,
                device_id=(tgt,), device_id_type=pl.DeviceIdType.MESH,
            )
            r.start()
            return r

        cA = [(z - s - 1) % 4 for s in range(3)] + [z]
        cB = [(y - s - 1) % 4 for s in range(3)] + [y]
        kA = [(y - 1 - i) % 4 for i in range(4)]
        kB = [(z - 1 - i) % 4 for i in range(4)]

        partA[...] = jnp.dot(
            x_ref[...], w_ref[:, pl.ds(0, H)],
            preferred_element_type=jnp.float32,
        )
        barrier_sem = pltpu.get_barrier_semaphore()
        for nbr in (right_z, left_z, right_y, left_y, px):
            pl.semaphore_signal(
                barrier_sem, inc=1,
                device_id=(nbr,), device_id_type=pl.DeviceIdType.MESH,
            )
        pl.semaphore_wait(barrier_sem, 5)

        aC = [rdma(partA.at[pl.ds(256 * cA[0] + 64 * kA[i], 64), :],
                   rA.at[0, pl.ds(64 * i, 64), :],
                   szA.at[i], rzA.at[i], right_z)
              for i in range(4)]
        partB[...] = jnp.dot(
            x_ref[...], w_ref[:, pl.ds(H, H)],
            preferred_element_type=jnp.float32,
        )
        bC = [rdma(partB.at[pl.ds(256 * kB[i] + 64 * cB[0], 64), :],
                   rB.at[0, pl.ds(64 * i, 64), :],
                   syB.at[i], ryB.at[i], right_y)
              for i in range(4)]

        for s in (1, 2):
            na = []
            for i in range(4):
                aC[i].wait()
                rA[s - 1, pl.ds(64 * i, 64), :] = (
                    rA[s - 1, pl.ds(64 * i, 64), :]
                    + partA[pl.ds(256 * cA[s] + 64 * kA[i], 64), :]
                )
                na.append(rdma(rA.at[s - 1, pl.ds(64 * i, 64), :],
                               rA.at[s, pl.ds(64 * i, 64), :],
                               szA.at[s * 4 + i], rzA.at[s * 4 + i],
                               right_z))
            nb = []
            for i in range(4):
                bC[i].wait()
                rB[s - 1, pl.ds(64 * i, 64), :] = (
                    rB[s - 1, pl.ds(64 * i, 64), :]
                    + partB[pl.ds(256 * kB[i] + 64 * cB[s], 64), :]
                )
                nb.append(rdma(rB.at[s - 1, pl.ds(64 * i, 64), :],
                               rB.at[s, pl.ds(64 * i, 64), :],
                               syB.at[s * 4 + i], ryB.at[s * 4 + i],
                               right_y))
            aC, bC = na, nb

        a2C = [None, None]
        for i in range(4):
            aC[i].wait()
            rA[2, pl.ds(64 * i, 64), :] = (
                rA[2, pl.ds(64 * i, 64), :]
                + partA[pl.ds(256 * cA[3] + 64 * kA[i], 64), :]
            )
            if i == 0:
                for q in range(2):
                    a2C[q] = rdma(rA.at[2, pl.ds(32 * q, 32), :],
                                  rA2.at[0, pl.ds(32 * q, 32), :],
                                  syA.at[q], ryA.at[q], right_y)
        b2C = [None, None]
        for i in range(4):
            bC[i].wait()
            rB[2, pl.ds(64 * i, 64), :] = (
                rB[2, pl.ds(64 * i, 64), :]
                + partB[pl.ds(256 * kB[i] + 64 * cB[3], 64), :]
            )
            if i == 0:
                for q in range(2):
                    b2C[q] = rdma(rB.at[2, pl.ds(32 * q, 32), :],
                                  rB2.at[0, pl.ds(32 * q, 32), :],
                                  szB.at[q], rzB.at[q], right_z)

        for s in (1, 2):
            na = [None, None]
            for q in range(2):
                a2C[q].wait()
                rA2[s - 1, pl.ds(32 * q, 32), :] = (
                    rA2[s - 1, pl.ds(32 * q, 32), :]
                    + rA[2, pl.ds(64 * s + 32 * q, 32), :]
                )
                na[q] = rdma(rA2.at[s - 1, pl.ds(32 * q, 32), :],
                             rA2.at[s, pl.ds(32 * q, 32), :],
                             syA.at[s * 2 + q], ryA.at[s * 2 + q], right_y)
            nb = [None, None]
            for q in range(2):
                b2C[q].wait()
                rB2[s - 1, pl.ds(32 * q, 32), :] = (
                    rB2[s - 1, pl.ds(32 * q, 32), :]
                    + rB[2, pl.ds(64 * s + 32 * q, 32), :]
                )
                nb[q] = rdma(rB2.at[s - 1, pl.ds(32 * q, 32), :],
                             rB2.at[s, pl.ds(32 * q, 32), :],
                             szB.at[s * 2 + q], rzB.at[s * 2 + q], right_z)
            a2C, b2C = na, nb

        u = 1 - t
        for q in range(2):
            a2C[q].wait()
            rA2[2, pl.ds(32 * q, 32), :] = (
                rA2[2, pl.ds(32 * q, 32), :]
                + rA[2, pl.ds(192 + 32 * q, 32), :]
            )
        xa = rdma(rA2.at[2, pl.ds(32 * u, 32), :], rX.at[0],
                  sX.at[0], rXs.at[0], px)
        for q in range(2):
            b2C[q].wait()
            rB2[2, pl.ds(32 * q, 32), :] = (
                rB2[2, pl.ds(32 * q, 32), :]
                + rB[2, pl.ds(192 + 32 * q, 32), :]
            )
        xb = rdma(rB2.at[2, pl.ds(32 * u, 32), :], rX.at[1],
                  sX.at[1], rXs.at[1], px)
        xa.wait()
        out_ref[:, pl.ds(0, H)] = _gelu(
            rA2[2, pl.ds(32 * t, 32), :] + rX[0, :, :]
        )
        xb.wait()
        out_ref[:, pl.ds(H, H)] = _gelu(
            rB2[2, pl.ds(32 * t, 32), :] + rX[1, :, :]
        )

    return pl.pallas_call(
        body,
        out_shape=jax.ShapeDtypeStruct((m_per, n), jnp.float32),
        in_specs=[
            pl.BlockSpec(memory_space=pltpu.VMEM),
            pl.BlockSpec(memory_space=pltpu.VMEM),
        ],
        out_specs=pl.BlockSpec(memory_space=pltpu.VMEM),
        scratch_shapes=[
            pltpu.VMEM((m, H), jnp.float32),
            pltpu.VMEM((m, H), jnp.float32),
            pltpu.VMEM((3, 256, H), jnp.float32),
            pltpu.VMEM((3, 256, H), jnp.float32),
            pltpu.VMEM((3, 64, H), jnp.float32),
            pltpu.VMEM((3, 64, H), jnp.float32),
            pltpu.VMEM((2, 32, H), jnp.float32),
            pltpu.SemaphoreType.DMA((12,)),
            pltpu.SemaphoreType.DMA((12,)),
            pltpu.SemaphoreType.DMA((12,)),
            pltpu.SemaphoreType.DMA((12,)),
            pltpu.SemaphoreType.DMA((6,)),
            pltpu.SemaphoreType.DMA((6,)),
            pltpu.SemaphoreType.DMA((6,)),
            pltpu.SemaphoreType.DMA((6,)),
            pltpu.SemaphoreType.DMA((2,)),
            pltpu.SemaphoreType.DMA((2,)),
        ],
        compiler_params=pltpu.CompilerParams(collective_id=0),
    )(x, w_mat)


# device time: 35568 ns/iter; 1.1131x vs baseline; 1.1131x over previous
import jax
import jax.numpy as jnp
from jax import lax
from jax.experimental import pallas as pl
from jax.experimental.pallas import tpu as pltpu

N_DEV = 32
H = 512


def _gelu(v):
    c = 0.7978845608028654
    return 0.5 * v * (1.0 + jnp.tanh(c * (v + 0.044715 * v * v * v)))


def kernel(x, w_mat):
    m, k_per = x.shape
    _, n = w_mat.shape
    m_per = m // N_DEV

    def body(x_ref, w_ref, out_ref, part, rA, rB, rXA, rXB,
             szA, rzA, syB, ryB, sDA, rDA, sDB, rDB):
        p = lax.axis_index("i")
        z = p // 8
        j = p % 8
        y = j // 2
        xc = (j + y) % 2
        t = p % 2
        mm = 2 * z + xc

        right_z = (p + 8) % N_DEV
        yn = (y + 1) % 4
        right_y = 8 * z + 2 * yn + (xc + yn) % 2

        colsA = pl.ds(0, H)
        colsB = pl.ds(H, H)
        qs = [pl.ds(256 * q, 256) for q in range(2)]

        def rdma(src, dst, ssem, rsem, tgt):
            r = pltpu.make_async_remote_copy(
                src_ref=src, dst_ref=dst, send_sem=ssem, recv_sem=rsem,
                device_id=(tgt,), device_id_type=pl.DeviceIdType.MESH,
            )
            r.start()
            return r

        cA = [(z - s - 1) % 4 for s in range(3)] + [z]
        cB = [(y - s - 1) % 4 for s in range(3)] + [y]

        part[:, colsA] = jnp.dot(
            x_ref[...], w_ref[:, colsA], preferred_element_type=jnp.float32
        )
        barrier_sem = pltpu.get_barrier_semaphore()
        for d in range(1, 8):
            jj = (j + d) % 8
            pl.semaphore_signal(
                barrier_sem, inc=1,
                device_id=(8 * z + jj,), device_id_type=pl.DeviceIdType.MESH,
            )
            mt = (mm + d) % 8
            tt = (mt % 2 + y) % 2
            pl.semaphore_signal(
                barrier_sem, inc=1,
                device_id=(8 * (mt // 2) + 2 * y + tt,),
                device_id_type=pl.DeviceIdType.MESH,
            )
        pl.semaphore_wait(barrier_sem, 14)

        aC = [rdma(part.at[pl.ds(cA[0] * 256, 256), qs[q]],
                   rA.at[0, :, qs[q]], szA.at[q], rzA.at[q], right_z)
              for q in range(2)]
        part[:, colsB] = jnp.dot(
            x_ref[...], w_ref[:, colsB], preferred_element_type=jnp.float32
        )
        bC = [rdma(part.at[pl.ds(256 * k + 64 * cB[0], 64), colsB],
                   rB.at[0, pl.ds(64 * k, 64), :],
                   syB.at[k], ryB.at[k], right_y)
              for k in range(4)]

        for s in (1, 2):
            na = []
            for q in range(2):
                aC[q].wait()
                rA[s - 1, :, qs[q]] = (
                    rA[s - 1, :, qs[q]]
                    + part[pl.ds(cA[s] * 256, 256), qs[q]]
                )
                na.append(rdma(rA.at[s - 1, :, qs[q]], rA.at[s, :, qs[q]],
                               szA.at[s * 2 + q], rzA.at[s * 2 + q],
                               right_z))
            nb = []
            for k in range(4):
                bC[k].wait()
                rB[s - 1, pl.ds(64 * k, 64), :] = (
                    rB[s - 1, pl.ds(64 * k, 64), :]
                    + part[pl.ds(256 * k + 64 * cB[s], 64), colsB]
                )
                nb.append(rdma(rB.at[s - 1, pl.ds(64 * k, 64), :],
                               rB.at[s, pl.ds(64 * k, 64), :],
                               syB.at[s * 4 + k], ryB.at[s * 4 + k],
                               right_y))
            aC, bC = na, nb

        for q in range(2):
            aC[q].wait()
            rA[2, :, qs[q]] = (
                rA[2, :, qs[q]] + part[pl.ds(cA[3] * 256, 256), qs[q]]
            )
        dA = [rdma(rA.at[2, pl.ds(32 * ((j + d) % 8), 32), :],
                   rXA.at[pl.ds(32 * j, 32), :],
                   sDA.at[d - 1], rDA.at[j], 8 * z + (j + d) % 8)
              for d in range(1, 8)]
        rXA[pl.ds(32 * j, 32), :] = rA[2, pl.ds(32 * j, 32), :]

        for k in range(4):
            bC[k].wait()
            rB[2, pl.ds(64 * k, 64), :] = (
                rB[2, pl.ds(64 * k, 64), :]
                + part[pl.ds(256 * k + 64 * cB[3], 64), colsB]
            )
        dB = []
        for d in range(1, 8):
            mt = (mm + d) % 8
            zt = mt // 2
            tt = (mt % 2 + y) % 2
            dB.append(rdma(rB.at[2, pl.ds(64 * zt + 32 * tt, 32), :],
                           rXB.at[pl.ds(32 * mm, 32), :],
                           sDB.at[d - 1], rDB.at[mm],
                           8 * zt + 2 * y + tt))
        rXB[pl.ds(32 * mm, 32), :] = rB[2, pl.ds(64 * z + 32 * t, 32), :]

        def recv_wait(buf, sems, slot, dummy_ssem):
            pltpu.make_async_remote_copy(
                src_ref=buf.at[pl.ds(32 * slot, 32), :],
                dst_ref=buf.at[pl.ds(32 * slot, 32), :],
                send_sem=dummy_ssem, recv_sem=sems.at[slot],
                device_id=(p,), device_id_type=pl.DeviceIdType.MESH,
            ).wait_recv()

        for d in range(1, 8):
            recv_wait(rXA, rDA, (j - d) % 8, sDA.at[0])
        accA = rXA[pl.ds(0, 32), :]
        for k in range(1, 8):
            accA = accA + rXA[pl.ds(32 * k, 32), :]
        out_ref[:, colsA] = _gelu(accA)

        for d in range(1, 8):
            recv_wait(rXB, rDB, (mm - d) % 8, sDB.at[0])
        accB = rXB[pl.ds(0, 32), :]
        for k in range(1, 8):
            accB = accB + rXB[pl.ds(32 * k, 32), :]
        out_ref[:, colsB] = _gelu(accB)

        for r in dA:
            r.wait_send()
        for r in dB:
            r.wait_send()

    return pl.pallas_call(
        body,
        out_shape=jax.ShapeDtypeStruct((m_per, n), jnp.float32),
        in_specs=[
            pl.BlockSpec(memory_space=pltpu.VMEM),
            pl.BlockSpec(memory_space=pltpu.VMEM),
        ],
        out_specs=pl.BlockSpec(memory_space=pltpu.VMEM),
        scratch_shapes=[
            pltpu.VMEM((m, n), jnp.float32),
            pltpu.VMEM((3, 256, H), jnp.float32),
            pltpu.VMEM((3, 256, H), jnp.float32),
            pltpu.VMEM((256, H), jnp.float32),
            pltpu.VMEM((256, H), jnp.float32),
            pltpu.SemaphoreType.DMA((6,)),
            pltpu.SemaphoreType.DMA((6,)),
            pltpu.SemaphoreType.DMA((12,)),
            pltpu.SemaphoreType.DMA((12,)),
            pltpu.SemaphoreType.DMA((7,)),
            pltpu.SemaphoreType.DMA((8,)),
            pltpu.SemaphoreType.DMA((7,)),
            pltpu.SemaphoreType.DMA((8,)),
        ],
        compiler_params=pltpu.CompilerParams(collective_id=0),
    )(x, w_mat)
